# Initial kernel scaffold; baseline (speedup 1.0000x reference)
import jax
import jax.numpy as jnp
from jax.experimental import pallas as pl


def kernel(char_logits, radical_logits, structure, stroke_count, stroke_types, radical_mask, structure_label, stroke_count_label, stroke_type_sig, W1, b1, W2, b2, reranker_weight):
    raise NotImplementedError("write your pallas kernel here")


# speedup vs baseline: 3.1773x; 3.1773x over previous
import functools

import jax
import jax.numpy as jnp
from jax.experimental import pallas as pl


_NEG = -3.0e38


def _prep_kernel(maskT_ref, sl_ref, sigT_ref, t1_ref, smt_ref, sgt_ref):
    maskT = maskT_ref[...]
    count = jnp.sum(maskT, axis=0, keepdims=True)
    t1_ref[...] = maskT / jnp.maximum(count, 1.0)

    sl = sl_ref[...]
    ns = smt_ref.shape[0]
    row_ids = jax.lax.broadcasted_iota(jnp.int32, (ns, sl.shape[1]), 0)
    smt_ref[...] = (row_ids == sl).astype(jnp.float32)

    sigT = sigT_ref[...]
    n2 = jnp.sum(sigT * sigT, axis=0, keepdims=True)
    nv = jnp.sqrt(n2)
    has = (nv > 1e-6).astype(jnp.float32)
    sgt_ref[...] = (sigT / jnp.maximum(nv, 1e-12)) * has


def _main_kernel(x_ref, rl_ref, st_ref, sc_ref, sty_ref,
                 t1_ref, maskT_ref, smt_ref, sgt_ref, sclf_ref,
                 w1_ref, b1_ref, w2t_ref, b2_ref, rw_ref, out_ref, *, k):
    f32 = jnp.float32
    x = x_ref[...]
    bb, c = x.shape

    rp = jax.nn.sigmoid(rl_ref[...])
    total = jnp.sum(rp, axis=1, keepdims=True)
    totclip = jnp.maximum(total, 1e-6)

    sp = jax.nn.softmax(st_ref[...], axis=1)

    sc = sc_ref[...]
    scm = jnp.max(sc, axis=1, keepdims=True)
    sc_iota = jax.lax.broadcasted_iota(jnp.int32, sc.shape, 1)
    pred = jnp.min(jnp.where(sc == scm, sc_iota, jnp.int32(2 ** 30)),
                   axis=1, keepdims=True).astype(f32)

    sty = sty_ref[...]
    stn = sty / jnp.maximum(
        jnp.sqrt(jnp.sum(sty * sty, axis=1, keepdims=True)), 1e-12)

    d1 = jnp.dot(rp, t1_ref[...], preferred_element_type=f32)
    d2 = jnp.dot(rp, maskT_ref[...], preferred_element_type=f32)
    d3 = jnp.dot(sp, smt_ref[...], preferred_element_type=f32)
    d5 = jnp.dot(stn, sgt_ref[...], preferred_element_type=f32)

    sclf = sclf_ref[...]

    m0 = jnp.max(x, axis=1, keepdims=True)
    se = jnp.sum(jnp.exp(x - m0), axis=1, keepdims=True)

    w1 = w1_ref[...]
    b1 = b1_ref[...]
    w2t = w2t_ref[...]
    b2 = b2_ref[0, 0]
    rw = rw_ref[0, 0]

    iota = jax.lax.broadcasted_iota(jnp.int32, (bb, c), 1)
    big = jnp.int32(2 ** 30)
    work = x
    out = x
    for _ in range(k):
        m = jnp.max(work, axis=1, keepdims=True)
        eq = work == m
        il = jnp.min(jnp.where(eq, iota, big), axis=1, keepdims=True)
        oneh = iota == il
        onehf = oneh.astype(f32)

        g1 = jnp.sum(d1 * onehf, axis=1, keepdims=True)
        g2 = jnp.sum(d2 * onehf, axis=1, keepdims=True)
        g3 = jnp.sum(d3 * onehf, axis=1, keepdims=True)
        g5 = jnp.sum(d5 * onehf, axis=1, keepdims=True)
        scl = jnp.sum(sclf * onehf, axis=1, keepdims=True)

        f1 = g1
        f2 = (total - g2) / totclip
        f4 = jnp.abs(pred - scl) * (1.0 / 29.0)
        f6 = jnp.exp(m - m0) / se

        h = (f1 * w1[0:1, :] + f2 * w1[1:2, :] + g3 * w1[2:3, :] +
             f4 * w1[3:4, :] + g5 * w1[4:5, :] + f6 * w1[5:6, :] + b1)
        h = jnp.maximum(h, 0.0)
        score = jnp.sum(h * w2t, axis=1, keepdims=True) + b2

        val = m + rw * score
        out = jnp.where(oneh, val, out)
        work = jnp.where(oneh, _NEG, work)

    out_ref[...] = out


@jax.jit
def kernel(char_logits, radical_logits, structure, stroke_count, stroke_types,
           radical_mask, structure_label, stroke_count_label, stroke_type_sig,
           W1, b1, W2, b2, reranker_weight):
    f32 = jnp.float32
    B, C = char_logits.shape
    K = min(20, C)
    R = radical_mask.shape[1]
    NS = structure.shape[1]
    NT = stroke_types.shape[1]
    H = W1.shape[1]

    maskT = radical_mask.T
    sigT = stroke_type_sig.T
    sl2 = structure_label.reshape(1, C)
    sclf = stroke_count_label.astype(f32).reshape(1, C)
    b1r = b1.reshape(1, H)
    w2t = W2.reshape(1, H)
    b2r = b2.reshape(1, 1)
    rwr = reranker_weight.reshape(1, 1)

    t1, smt, sgt = pl.pallas_call(
        _prep_kernel,
        out_shape=[
            jax.ShapeDtypeStruct((R, C), f32),
            jax.ShapeDtypeStruct((NS, C), f32),
            jax.ShapeDtypeStruct((NT, C), f32),
        ],
    )(maskT, sl2, sigT)

    BB = 128
    grid = (B // BB,)

    def rows(shape1):
        return pl.BlockSpec((BB, shape1), lambda i: (i, 0))

    def full(s0, s1):
        return pl.BlockSpec((s0, s1), lambda i: (0, 0))

    out = pl.pallas_call(
        functools.partial(_main_kernel, k=K),
        grid=grid,
        in_specs=[
            rows(C),
            rows(radical_logits.shape[1]),
            rows(NS),
            rows(stroke_count.shape[1]),
            rows(NT),
            full(R, C),
            full(R, C),
            full(NS, C),
            full(NT, C),
            full(1, C),
            full(6, H),
            full(1, H),
            full(1, H),
            full(1, 1),
            full(1, 1),
        ],
        out_specs=rows(C),
        out_shape=jax.ShapeDtypeStruct((B, C), f32),
    )(char_logits, radical_logits, structure, stroke_count, stroke_types,
      t1, maskT, smt, sgt, sclf, W1, b1r, w2t, b2r, rwr)
    return out


# speedup vs baseline: 3.3141x; 1.0431x over previous
import functools

import jax
import jax.numpy as jnp
from jax import lax
from jax.experimental import pallas as pl
from jax.experimental.pallas import tpu as pltpu
from jax.experimental.pallas import tpu_sc as plsc


_NEG = -3.0e38
_K = 20
_KPAD = 32
_TW = 256
_RPW = 224
_CHROWS = 8


def _table_kernel(mask_ref, scl_ref, sl_ref, sig_ref, tbl_ref):
    f32 = jnp.float32
    mask = mask_ref[...]
    c, r = mask.shape
    count = jnp.sum(mask, axis=1, keepdims=True)
    countinv = 1.0 / jnp.maximum(count, 1.0)

    sig = sig_ref[...]
    nv = jnp.sqrt(jnp.sum(sig * sig, axis=1, keepdims=True))
    has = (nv > 1e-6).astype(f32)
    signorm = (sig / jnp.maximum(nv, 1e-12)) * has

    tbl_ref[:, 0:r] = mask
    tbl_ref[:, r:224] = jnp.zeros((c, 224 - r), f32)
    tbl_ref[:, 224:225] = countinv
    tbl_ref[:, 225:226] = scl_ref[...].astype(f32)
    tbl_ref[:, 226:227] = sl_ref[...].astype(f32)
    tbl_ref[:, 227:240] = jnp.zeros((c, 13), f32)
    tbl_ref[:, 240:246] = signorm
    tbl_ref[:, 246:256] = jnp.zeros((c, 10), f32)


def _rowprep_kernel(x_ref, rl_ref, st_ref, sc_ref, sty_ref,
                    ti_ref, tv_ref, rp_ref, sp_ref, stn_ref, qs_ref):
    f32 = jnp.float32
    x = x_ref[...]
    bb, c = x.shape

    rp = jax.nn.sigmoid(rl_ref[...])
    total = jnp.sum(rp, axis=1, keepdims=True)
    inv_totclip = 1.0 / jnp.maximum(total, 1e-6)
    rp_ref[:, 0:rp.shape[1]] = rp
    rp_ref[:, rp.shape[1]:] = jnp.zeros((bb, _RPW - rp.shape[1]), f32)

    sp = jax.nn.softmax(st_ref[...], axis=1)
    sp_ref[:, 0:sp.shape[1]] = sp
    sp_ref[:, sp.shape[1]:] = jnp.zeros((bb, 16 - sp.shape[1]), f32)

    sc = sc_ref[...]
    scm = jnp.max(sc, axis=1, keepdims=True)
    sc_iota = lax.broadcasted_iota(jnp.int32, sc.shape, 1)
    pred = jnp.min(jnp.where(sc == scm, sc_iota, jnp.int32(2 ** 30)),
                   axis=1, keepdims=True).astype(f32)

    sty = sty_ref[...]
    stn = sty / jnp.maximum(
        jnp.sqrt(jnp.sum(sty * sty, axis=1, keepdims=True)), 1e-12)
    stn_ref[:, 0:stn.shape[1]] = stn
    stn_ref[:, stn.shape[1]:] = jnp.zeros((bb, 16 - stn.shape[1]), f32)

    m0 = jnp.max(x, axis=1, keepdims=True)
    se = jnp.sum(jnp.exp(x - m0), axis=1, keepdims=True)
    qs_ref[...] = jnp.concatenate(
        [total, inv_totclip, pred, m0, 1.0 / se,
         jnp.zeros((bb, 11), f32)], axis=1)

    nl = 128
    ns_ = (c + nl - 1) // nl
    ninf = jnp.float32(-jnp.inf)
    lane = lax.broadcasted_iota(jnp.int32, (bb, nl), 1)
    m1 = jnp.full((bb, nl), ninf, f32)
    m2, m3, m4 = m1, m1, m1
    g1 = jnp.zeros((bb, nl), jnp.int32)
    g2, g3, g4 = g1, g1, g1
    for s in range(ns_):
        lo = s * nl
        w = min(nl, c - lo)
        xs = x[:, lo:lo + w]
        if w < nl:
            xs = jnp.concatenate(
                [xs, jnp.full((bb, nl - w), ninf, f32)], axis=1)
        gs = lane + lo
        b1 = xs > m1
        b2 = (xs > m2) & ~b1
        b3 = (xs > m3) & ~b1 & ~b2
        b4 = (xs > m4) & ~b1 & ~b2 & ~b3
        b12 = b1 | b2
        b123 = b12 | b3
        m4 = jnp.where(b123, m3, jnp.where(b4, xs, m4))
        g4 = jnp.where(b123, g3, jnp.where(b4, gs, g4))
        m3 = jnp.where(b12, m2, jnp.where(b3, xs, m3))
        g3 = jnp.where(b12, g2, jnp.where(b3, gs, g3))
        m2 = jnp.where(b1, m1, jnp.where(b2, xs, m2))
        g2 = jnp.where(b1, g1, jnp.where(b2, gs, g2))
        m1 = jnp.where(b1, xs, m1)
        g1 = jnp.where(b1, gs, g1)

    big = jnp.int32(2 ** 30)
    cur, gc = m1, g1
    lvl = jnp.zeros((bb, nl), jnp.int32)
    failv = jnp.zeros((bb, nl), jnp.bool_)
    tis, tvs = [], []
    for _ in range(_K):
        m = jnp.max(cur, axis=1, keepdims=True)
        sel = cur == m
        gsel = jnp.min(jnp.where(sel, gc, big), axis=1, keepdims=True)
        tis.append(gsel)
        tvs.append(m)
        win = sel & (gc == gsel)
        failv = failv | (win & (lvl >= 3))
        ncur = jnp.where(lvl == 0, m2,
                         jnp.where(lvl == 1, m3,
                                   jnp.where(lvl == 2, m4, ninf)))
        ngc = jnp.where(lvl == 0, g2,
                        jnp.where(lvl == 1, g3,
                                  jnp.where(lvl == 2, g4, 0)))
        cur = jnp.where(win, ncur, cur)
        gc = jnp.where(win, ngc, gc)
        lvl = lvl + win.astype(jnp.int32)
    row_id = (pl.program_id(0) * bb
              + lax.broadcasted_iota(jnp.int32, (bb, _KPAD - _K), 0))
    pad_ids = ((row_id * (_KPAD - _K)
                + lax.broadcasted_iota(jnp.int32, (bb, _KPAD - _K), 1))
               % jnp.int32(c))
    ti_ref[...] = jnp.concatenate(tis + [pad_ids], axis=1)
    tv_ref[...] = jnp.concatenate(tvs + [jnp.zeros((bb, _KPAD - _K), f32)],
                                  axis=1)

    @pl.when(jnp.any(failv))
    def _redo():
        iota = lax.broadcasted_iota(jnp.int32, (bb, c), 1)
        work = x
        for k in range(_K):
            mm = jnp.max(work, axis=1, keepdims=True)
            eq = work == mm
            il = jnp.min(jnp.where(eq, iota, big), axis=1, keepdims=True)
            oneh = iota == il
            ti_ref[:, k:k + 1] = il
            tv_ref[:, k:k + 1] = mm
            work = jnp.where(oneh, ninf, work)


def _sc_kernel(char_hbm, ti_hbm, tv_hbm, rp_hbm, sp_hbm, stn_hbm, qs_hbm,
               tbl_hbm, wc_hbm, out_hbm,
               chunk_v, ti_v, tv_v, rp_v, sp_v, stn_v, qs_v, gath_v,
               vals_v, wc_v, red_v, sem, *, n_rows, c_dim):
    f32 = jnp.float32
    nc = 2
    wid = lax.axis_index("s") * nc + lax.axis_index("c")
    rows_per_w = n_rows // 32
    nchunks = rows_per_w // _CHROWS
    chunk_words = _CHROWS * c_dim
    ncand = _CHROWS * _KPAD

    pltpu.sync_copy(wc_hbm, wc_v)

    iota16 = lax.iota(jnp.int32, 16)
    mask0 = iota16 == 0
    mask4 = iota16 < (_K - 16)
    cv = wc_v[pl.ds(512, 16)]
    b2s = cv[0]
    rws = cv[1]
    rot_idx = [(iota16 + sh) & 15 for sh in (8, 4, 2, 1)]

    def vsum16(v):
        for idx in rot_idx:
            red_v[...] = v
            v = v + plsc.load_gather(red_v, [idx])
        return v

    def do_chunk(ch, carry):
        row0 = wid * rows_per_w + ch * _CHROWS
        base = row0 * c_dim
        pltpu.sync_copy(char_hbm.at[pl.ds(base, chunk_words)], chunk_v)
        pltpu.sync_copy(ti_hbm.at[pl.ds(row0 * _KPAD, ncand)], ti_v)
        pltpu.sync_copy(tv_hbm.at[pl.ds(row0 * _KPAD, ncand)], tv_v)
        pltpu.sync_copy(rp_hbm.at[pl.ds(row0 * _RPW, _CHROWS * _RPW)], rp_v)
        pltpu.sync_copy(sp_hbm.at[pl.ds(row0 * 16, _CHROWS * 16)], sp_v)
        pltpu.sync_copy(stn_hbm.at[pl.ds(row0 * 16, _CHROWS * 16)], stn_v)
        pltpu.sync_copy(qs_hbm.at[pl.ds(row0 * 16, _CHROWS * 16)], qs_v)

        cp1 = pltpu.async_copy(tbl_hbm.at[ti_v.at[pl.ds(0, 128)]],
                               gath_v.at[pl.ds(0, 128), :], sem)
        cp2 = pltpu.async_copy(tbl_hbm.at[ti_v.at[pl.ds(128, 128)]],
                               gath_v.at[pl.ds(128, 128), :], sem)
        cp1.wait()
        cp2.wait()

        def do_cand(c_i, carry2):
            r8 = c_i // _K
            slot = r8 * _KPAD + (c_i % _K)
            rbase = r8 * _RPW
            gv = gath_v.at[slot]

            acc = jnp.zeros((16,), f32)
            def dot_step(j, a):
                return a + (rp_v[pl.ds(rbase + j * 16, 16)]
                            * gv[pl.ds(j * 16, 16)])
            acc = lax.fori_loop(0, _RPW // 16, dot_step, acc)
            det = vsum16(acc)[0]

            sv = gv[pl.ds(224, 16)]
            countinv = sv[0]
            sclf = sv[1]
            slf = sv[2].astype(jnp.int32)
            sig = gv[pl.ds(240, 16)]
            stn16 = stn_v[pl.ds(r8 * 16, 16)]
            cos = vsum16(sig * stn16)[0]

            qv = qs_v[pl.ds(r8 * 16, 16)]
            total = qv[0]
            inv_totclip = qv[1]
            pred = qv[2]
            m0 = qv[3]
            inv_se = qv[4]
            f3v = plsc.load_gather(
                sp_v, [jnp.full((16,), r8 * 16 + slf, jnp.int32)])
            tvv = plsc.load_gather(
                tv_v, [jnp.full((16,), slot, jnp.int32)])
            tval = tvv[0]

            f1 = det * countinv
            f2 = (total - det) * inv_totclip
            f4 = jnp.abs(pred - sclf) * (1.0 / 29.0)
            f5 = cos
            f6v = jnp.exp(tvv - m0) * inv_se

            fs = [jnp.full((16,), f1, f32), jnp.full((16,), f2, f32),
                  f3v, jnp.full((16,), f4, f32),
                  jnp.full((16,), f5, f32), f6v]
            sacc = jnp.zeros((16,), f32)
            for t in range(4):
                h = wc_v[pl.ds(384 + t * 16, 16)]
                for j in range(6):
                    h = h + fs[j] * wc_v[pl.ds(j * 64 + t * 16, 16)]
                h = jnp.maximum(h, 0.0)
                sacc = sacc + h * wc_v[pl.ds(448 + t * 16, 16)]
            score = vsum16(sacc)[0] + b2s
            val = tval + rws * score

            plsc.store_scatter(vals_v, [jnp.full((16,), slot, jnp.int32)],
                               jnp.full((16,), val, f32), mask=mask0)
            return carry2

        lax.fori_loop(0, _CHROWS * _K, do_cand, 0)

        for g in range(_CHROWS):
            pos1 = ti_v[pl.ds(g * _KPAD, 16)] + g * c_dim
            plsc.store_scatter(chunk_v, [pos1], vals_v[pl.ds(g * _KPAD, 16)])
            pos2 = ti_v[pl.ds(g * _KPAD + 16, 16)] + g * c_dim
            plsc.store_scatter(chunk_v, [pos2],
                               vals_v[pl.ds(g * _KPAD + 16, 16)], mask=mask4)

        pltpu.sync_copy(chunk_v, out_hbm.at[pl.ds(base, chunk_words)])
        return carry

    lax.fori_loop(0, nchunks, do_chunk, 0)


@jax.jit
def kernel(char_logits, radical_logits, structure, stroke_count, stroke_types,
           radical_mask, structure_label, stroke_count_label, stroke_type_sig,
           W1, b1, W2, b2, reranker_weight):
    f32 = jnp.float32
    B, C = char_logits.shape
    R = radical_mask.shape[1]
    NS = structure.shape[1]
    NT = stroke_types.shape[1]
    H = W1.shape[1]

    tbl = pl.pallas_call(
        _table_kernel,
        out_shape=jax.ShapeDtypeStruct((C, _TW), f32),
    )(radical_mask, stroke_count_label.reshape(C, 1),
      structure_label.reshape(C, 1), stroke_type_sig)

    BB = 128
    ti, tv, rp, sp, stn, qs = pl.pallas_call(
        _rowprep_kernel,
        grid=(B // BB,),
        in_specs=[
            pl.BlockSpec((BB, C), lambda i: (i, 0)),
            pl.BlockSpec((BB, R), lambda i: (i, 0)),
            pl.BlockSpec((BB, NS), lambda i: (i, 0)),
            pl.BlockSpec((BB, stroke_count.shape[1]), lambda i: (i, 0)),
            pl.BlockSpec((BB, NT), lambda i: (i, 0)),
        ],
        out_specs=[
            pl.BlockSpec((BB, _KPAD), lambda i: (i, 0)),
            pl.BlockSpec((BB, _KPAD), lambda i: (i, 0)),
            pl.BlockSpec((BB, _RPW), lambda i: (i, 0)),
            pl.BlockSpec((BB, 16), lambda i: (i, 0)),
            pl.BlockSpec((BB, 16), lambda i: (i, 0)),
            pl.BlockSpec((BB, 16), lambda i: (i, 0)),
        ],
        out_shape=[
            jax.ShapeDtypeStruct((B, _KPAD), jnp.int32),
            jax.ShapeDtypeStruct((B, _KPAD), f32),
            jax.ShapeDtypeStruct((B, _RPW), f32),
            jax.ShapeDtypeStruct((B, 16), f32),
            jax.ShapeDtypeStruct((B, 16), f32),
            jax.ShapeDtypeStruct((B, 16), f32),
        ],
    )(char_logits, radical_logits, structure, stroke_count, stroke_types)

    wc = jnp.concatenate([
        W1.reshape(-1), b1.reshape(-1), W2.reshape(-1),
        b2.reshape(-1), reranker_weight.reshape(-1),
        jnp.zeros((14,), f32)])

    ncand = _CHROWS * _KPAD
    mesh = plsc.VectorSubcoreMesh(core_axis_name="c", subcore_axis_name="s")
    out_flat = pl.kernel(
        functools.partial(_sc_kernel, n_rows=B, c_dim=C),
        mesh=mesh,
        compiler_params=pltpu.CompilerParams(
            needs_layout_passes=False, use_tc_tiling_on_sc=False),
        out_type=jax.ShapeDtypeStruct((B * C,), f32),
        scratch_types=[
            pltpu.VMEM((_CHROWS * C,), f32),
            pltpu.VMEM((ncand,), jnp.int32),
            pltpu.VMEM((ncand,), f32),
            pltpu.VMEM((_CHROWS * _RPW,), f32),
            pltpu.VMEM((_CHROWS * 16,), f32),
            pltpu.VMEM((_CHROWS * 16,), f32),
            pltpu.VMEM((_CHROWS * 16,), f32),
            pltpu.VMEM((ncand, _TW), f32),
            pltpu.VMEM((ncand,), f32),
            pltpu.VMEM((528,), f32),
            pltpu.VMEM((16,), f32),
            pltpu.SemaphoreType.DMA,
        ],
    )(char_logits.reshape(-1), ti.reshape(-1), tv.reshape(-1),
      rp.reshape(-1), sp.reshape(-1), stn.reshape(-1), qs.reshape(-1),
      tbl, wc)
    return out_flat.reshape(B, C)


# speedup vs baseline: 3.4119x; 1.0295x over previous
import functools

import jax
import jax.numpy as jnp
from jax import lax
from jax.experimental import pallas as pl
from jax.experimental.pallas import tpu as pltpu
from jax.experimental.pallas import tpu_sc as plsc


_NEG = -3.0e38
_K = 20
_KPAD = 32
_TW = 256
_RPW = 224
_CHROWS = 8


def _table_kernel(mask_ref, scl_ref, sl_ref, sig_ref, tbl_ref):
    f32 = jnp.float32
    mask = mask_ref[...]
    c, r = mask.shape
    count = jnp.sum(mask, axis=1, keepdims=True)
    countinv = 1.0 / jnp.maximum(count, 1.0)

    sig = sig_ref[...]
    nv = jnp.sqrt(jnp.sum(sig * sig, axis=1, keepdims=True))
    has = (nv > 1e-6).astype(f32)
    signorm = (sig / jnp.maximum(nv, 1e-12)) * has

    tbl_ref[:, 0:r] = mask
    tbl_ref[:, r:224] = jnp.zeros((c, 224 - r), f32)
    tbl_ref[:, 224:225] = countinv
    tbl_ref[:, 225:226] = scl_ref[...].astype(f32)
    tbl_ref[:, 226:227] = sl_ref[...].astype(f32)
    tbl_ref[:, 227:240] = jnp.zeros((c, 13), f32)
    tbl_ref[:, 240:246] = signorm
    tbl_ref[:, 246:256] = jnp.zeros((c, 10), f32)


def _rowprep_kernel(x_ref, rl_ref, st_ref, sc_ref, sty_ref,
                    ti_ref, tv_ref, rp_ref, sp_ref, stn_ref, qs_ref):
    f32 = jnp.float32
    x = x_ref[...]
    bb, c = x.shape

    rp = jax.nn.sigmoid(rl_ref[...])
    total = jnp.sum(rp, axis=1, keepdims=True)
    inv_totclip = 1.0 / jnp.maximum(total, 1e-6)
    rp_ref[:, 0:rp.shape[1]] = rp
    rp_ref[:, rp.shape[1]:] = jnp.zeros((bb, _RPW - rp.shape[1]), f32)

    sp = jax.nn.softmax(st_ref[...], axis=1)
    sp_ref[:, 0:sp.shape[1]] = sp
    sp_ref[:, sp.shape[1]:] = jnp.zeros((bb, 16 - sp.shape[1]), f32)

    sc = sc_ref[...]
    scm = jnp.max(sc, axis=1, keepdims=True)
    sc_iota = lax.broadcasted_iota(jnp.int32, sc.shape, 1)
    pred = jnp.min(jnp.where(sc == scm, sc_iota, jnp.int32(2 ** 30)),
                   axis=1, keepdims=True).astype(f32)

    sty = sty_ref[...]
    stn = sty / jnp.maximum(
        jnp.sqrt(jnp.sum(sty * sty, axis=1, keepdims=True)), 1e-12)
    stn_ref[:, 0:stn.shape[1]] = stn
    stn_ref[:, stn.shape[1]:] = jnp.zeros((bb, 16 - stn.shape[1]), f32)

    m0 = jnp.max(x, axis=1, keepdims=True)
    se = jnp.sum(jnp.exp(x - m0), axis=1, keepdims=True)
    qs_ref[...] = jnp.concatenate(
        [total, inv_totclip, pred, m0, 1.0 / se,
         jnp.zeros((bb, 11), f32)], axis=1)

    nl = 128
    ns_ = (c + nl - 1) // nl
    ninf = jnp.float32(-jnp.inf)
    lane = lax.broadcasted_iota(jnp.int32, (bb, nl), 1)
    m1 = jnp.full((bb, nl), ninf, f32)
    m2, m3, m4 = m1, m1, m1
    g1 = jnp.zeros((bb, nl), jnp.int32)
    g2, g3, g4 = g1, g1, g1
    for s in range(ns_):
        lo = s * nl
        w = min(nl, c - lo)
        xs = x[:, lo:lo + w]
        if w < nl:
            xs = jnp.concatenate(
                [xs, jnp.full((bb, nl - w), ninf, f32)], axis=1)
        gs = lane + lo
        b1 = xs > m1
        b2 = (xs > m2) & ~b1
        b3 = (xs > m3) & ~b1 & ~b2
        b4 = (xs > m4) & ~b1 & ~b2 & ~b3
        b12 = b1 | b2
        b123 = b12 | b3
        m4 = jnp.where(b123, m3, jnp.where(b4, xs, m4))
        g4 = jnp.where(b123, g3, jnp.where(b4, gs, g4))
        m3 = jnp.where(b12, m2, jnp.where(b3, xs, m3))
        g3 = jnp.where(b12, g2, jnp.where(b3, gs, g3))
        m2 = jnp.where(b1, m1, jnp.where(b2, xs, m2))
        g2 = jnp.where(b1, g1, jnp.where(b2, gs, g2))
        m1 = jnp.where(b1, xs, m1)
        g1 = jnp.where(b1, gs, g1)

    big = jnp.int32(2 ** 30)
    cur, gc = m1, g1
    lvl = jnp.zeros((bb, nl), jnp.int32)
    failv = jnp.zeros((bb, nl), jnp.bool_)
    tis, tvs = [], []
    for _ in range(_K):
        m = jnp.max(cur, axis=1, keepdims=True)
        sel = cur == m
        gsel = jnp.min(jnp.where(sel, gc, big), axis=1, keepdims=True)
        tis.append(gsel)
        tvs.append(m)
        win = sel & (gc == gsel)
        failv = failv | (win & (lvl >= 3))
        ncur = jnp.where(lvl == 0, m2,
                         jnp.where(lvl == 1, m3,
                                   jnp.where(lvl == 2, m4, ninf)))
        ngc = jnp.where(lvl == 0, g2,
                        jnp.where(lvl == 1, g3,
                                  jnp.where(lvl == 2, g4, 0)))
        cur = jnp.where(win, ncur, cur)
        gc = jnp.where(win, ngc, gc)
        lvl = lvl + win.astype(jnp.int32)
    row_id = (pl.program_id(0) * bb
              + lax.broadcasted_iota(jnp.int32, (bb, _KPAD - _K), 0))
    pad_ids = ((row_id * (_KPAD - _K)
                + lax.broadcasted_iota(jnp.int32, (bb, _KPAD - _K), 1))
               % jnp.int32(c))
    ti_ref[...] = jnp.concatenate(tis + [pad_ids], axis=1)
    tv_ref[...] = jnp.concatenate(tvs + [jnp.zeros((bb, _KPAD - _K), f32)],
                                  axis=1)

    @pl.when(jnp.any(failv))
    def _redo():
        iota = lax.broadcasted_iota(jnp.int32, (bb, c), 1)
        work = x
        for k in range(_K):
            mm = jnp.max(work, axis=1, keepdims=True)
            eq = work == mm
            il = jnp.min(jnp.where(eq, iota, big), axis=1, keepdims=True)
            oneh = iota == il
            ti_ref[:, k:k + 1] = il
            tv_ref[:, k:k + 1] = mm
            work = jnp.where(oneh, ninf, work)


def _sc_kernel(char_hbm, ti_hbm, tv_hbm, rp_hbm, sp_hbm, stn_hbm, qs_hbm,
               tbl_hbm, wc_hbm, out_hbm,
               chunk_v, ti_v, tv_v, rp_v, sp_v, stn_v, qs_v, gath_v,
               vals_v, wc_v, red_v, sem, *, n_rows, c_dim):
    f32 = jnp.float32
    nc = 2
    wid = lax.axis_index("s") * nc + lax.axis_index("c")
    rows_per_w = n_rows // 32
    nchunks = rows_per_w // _CHROWS
    chunk_words = _CHROWS * c_dim
    ncand = _CHROWS * _KPAD

    pltpu.sync_copy(wc_hbm, wc_v)

    iota16 = lax.iota(jnp.int32, 16)
    mask0 = iota16 == 0
    mask4 = iota16 < (_K - 16)
    cv = wc_v[pl.ds(512, 16)]
    b2s = cv[0]
    rws = cv[1]
    rot_idx = [(iota16 + sh) & 15 for sh in (8, 4, 2, 1)]

    def vsum16(v, base=0):
        for idx in rot_idx:
            red_v[pl.ds(base, 16)] = v
            v = v + plsc.load_gather(red_v.at[pl.ds(base, 16)], [idx])
        return v

    def vsum16x2(va, vb):
        for idx in rot_idx:
            red_v[pl.ds(0, 16)] = va
            red_v[pl.ds(16, 16)] = vb
            va = va + plsc.load_gather(red_v.at[pl.ds(0, 16)], [idx])
            vb = vb + plsc.load_gather(red_v.at[pl.ds(16, 16)], [idx])
        return va, vb

    def do_chunk(ch, carry):
        row0 = wid * rows_per_w + ch * _CHROWS
        base = row0 * c_dim
        pltpu.sync_copy(char_hbm.at[pl.ds(base, chunk_words)], chunk_v)
        pltpu.sync_copy(ti_hbm.at[pl.ds(row0 * _KPAD, ncand)], ti_v)
        pltpu.sync_copy(tv_hbm.at[pl.ds(row0 * _KPAD, ncand)], tv_v)
        pltpu.sync_copy(rp_hbm.at[pl.ds(row0 * _RPW, _CHROWS * _RPW)], rp_v)
        pltpu.sync_copy(sp_hbm.at[pl.ds(row0 * 16, _CHROWS * 16)], sp_v)
        pltpu.sync_copy(stn_hbm.at[pl.ds(row0 * 16, _CHROWS * 16)], stn_v)
        pltpu.sync_copy(qs_hbm.at[pl.ds(row0 * 16, _CHROWS * 16)], qs_v)

        cp1 = pltpu.async_copy(tbl_hbm.at[ti_v.at[pl.ds(0, 128)]],
                               gath_v.at[pl.ds(0, 128), :], sem)
        cp2 = pltpu.async_copy(tbl_hbm.at[ti_v.at[pl.ds(128, 128)]],
                               gath_v.at[pl.ds(128, 128), :], sem)
        cp1.wait()
        cp2.wait()

        def do_row(r8, carry2):
            rbase = r8 * _RPW
            qv = qs_v[pl.ds(r8 * 16, 16)]
            total = qv[0]
            inv_totclip = qv[1]
            pred = qv[2]
            m0 = qv[3]
            inv_se = qv[4]
            stn16 = stn_v[pl.ds(r8 * 16, 16)]

            def do_cand(o, carry3):
                slot = r8 * _KPAD + o
                gv = gath_v.at[slot]
                accs = [rp_v[pl.ds(rbase + j * 16, 16)] * gv[pl.ds(j * 16, 16)]
                        for j in range(4)]
                for j in range(4, _RPW // 16):
                    accs[j % 4] = accs[j % 4] + (
                        rp_v[pl.ds(rbase + j * 16, 16)] * gv[pl.ds(j * 16, 16)])
                acc = (accs[0] + accs[1]) + (accs[2] + accs[3])

                sv = gv[pl.ds(224, 16)]
                countinv = sv[0]
                sclf = sv[1]
                slf = sv[2].astype(jnp.int32)
                sig = gv[pl.ds(240, 16)]
                detv, cosv = vsum16x2(acc, sig * stn16)
                det = detv[0]
                cos = cosv[0]

                f3v = plsc.load_gather(
                    sp_v, [jnp.full((16,), r8 * 16 + slf, jnp.int32)])
                tvv = plsc.load_gather(
                    tv_v, [jnp.full((16,), slot, jnp.int32)])
                tval = tvv[0]

                f1 = det * countinv
                f2 = (total - det) * inv_totclip
                f4 = jnp.abs(pred - sclf) * (1.0 / 29.0)
                f6v = jnp.exp(tvv - m0) * inv_se

                fs = [jnp.full((16,), f1, f32), jnp.full((16,), f2, f32),
                      f3v, jnp.full((16,), f4, f32),
                      jnp.full((16,), cos, f32), f6v]
                sacc = jnp.zeros((16,), f32)
                for t in range(4):
                    h = wc_v[pl.ds(384 + t * 16, 16)]
                    for j in range(6):
                        h = h + fs[j] * wc_v[pl.ds(j * 64 + t * 16, 16)]
                    h = jnp.maximum(h, 0.0)
                    sacc = sacc + h * wc_v[pl.ds(448 + t * 16, 16)]
                score = vsum16(sacc, 32)[0] + b2s
                val = tval + rws * score

                plsc.store_scatter(vals_v, [jnp.full((16,), slot, jnp.int32)],
                                   jnp.full((16,), val, f32), mask=mask0)
                return carry3

            lax.fori_loop(0, _K, do_cand, 0)
            return carry2

        lax.fori_loop(0, _CHROWS, do_row, 0)

        for g in range(_CHROWS):
            pos1 = ti_v[pl.ds(g * _KPAD, 16)] + g * c_dim
            plsc.store_scatter(chunk_v, [pos1], vals_v[pl.ds(g * _KPAD, 16)])
            pos2 = ti_v[pl.ds(g * _KPAD + 16, 16)] + g * c_dim
            plsc.store_scatter(chunk_v, [pos2],
                               vals_v[pl.ds(g * _KPAD + 16, 16)], mask=mask4)

        pltpu.sync_copy(chunk_v, out_hbm.at[pl.ds(base, chunk_words)])
        return carry

    lax.fori_loop(0, nchunks, do_chunk, 0)


@jax.jit
def kernel(char_logits, radical_logits, structure, stroke_count, stroke_types,
           radical_mask, structure_label, stroke_count_label, stroke_type_sig,
           W1, b1, W2, b2, reranker_weight):
    f32 = jnp.float32
    B, C = char_logits.shape
    R = radical_mask.shape[1]
    NS = structure.shape[1]
    NT = stroke_types.shape[1]
    H = W1.shape[1]

    tbl = pl.pallas_call(
        _table_kernel,
        out_shape=jax.ShapeDtypeStruct((C, _TW), f32),
    )(radical_mask, stroke_count_label.reshape(C, 1),
      structure_label.reshape(C, 1), stroke_type_sig)

    BB = 128
    ti, tv, rp, sp, stn, qs = pl.pallas_call(
        _rowprep_kernel,
        grid=(B // BB,),
        in_specs=[
            pl.BlockSpec((BB, C), lambda i: (i, 0)),
            pl.BlockSpec((BB, R), lambda i: (i, 0)),
            pl.BlockSpec((BB, NS), lambda i: (i, 0)),
            pl.BlockSpec((BB, stroke_count.shape[1]), lambda i: (i, 0)),
            pl.BlockSpec((BB, NT), lambda i: (i, 0)),
        ],
        out_specs=[
            pl.BlockSpec((BB, _KPAD), lambda i: (i, 0)),
            pl.BlockSpec((BB, _KPAD), lambda i: (i, 0)),
            pl.BlockSpec((BB, _RPW), lambda i: (i, 0)),
            pl.BlockSpec((BB, 16), lambda i: (i, 0)),
            pl.BlockSpec((BB, 16), lambda i: (i, 0)),
            pl.BlockSpec((BB, 16), lambda i: (i, 0)),
        ],
        out_shape=[
            jax.ShapeDtypeStruct((B, _KPAD), jnp.int32),
            jax.ShapeDtypeStruct((B, _KPAD), f32),
            jax.ShapeDtypeStruct((B, _RPW), f32),
            jax.ShapeDtypeStruct((B, 16), f32),
            jax.ShapeDtypeStruct((B, 16), f32),
            jax.ShapeDtypeStruct((B, 16), f32),
        ],
    )(char_logits, radical_logits, structure, stroke_count, stroke_types)

    wc = jnp.concatenate([
        W1.reshape(-1), b1.reshape(-1), W2.reshape(-1),
        b2.reshape(-1), reranker_weight.reshape(-1),
        jnp.zeros((14,), f32)])

    ncand = _CHROWS * _KPAD
    mesh = plsc.VectorSubcoreMesh(core_axis_name="c", subcore_axis_name="s")
    out_flat = pl.kernel(
        functools.partial(_sc_kernel, n_rows=B, c_dim=C),
        mesh=mesh,
        compiler_params=pltpu.CompilerParams(
            needs_layout_passes=False, use_tc_tiling_on_sc=False),
        out_type=jax.ShapeDtypeStruct((B * C,), f32),
        scratch_types=[
            pltpu.VMEM((_CHROWS * C,), f32),
            pltpu.VMEM((ncand,), jnp.int32),
            pltpu.VMEM((ncand,), f32),
            pltpu.VMEM((_CHROWS * _RPW,), f32),
            pltpu.VMEM((_CHROWS * 16,), f32),
            pltpu.VMEM((_CHROWS * 16,), f32),
            pltpu.VMEM((_CHROWS * 16,), f32),
            pltpu.VMEM((ncand, _TW), f32),
            pltpu.VMEM((ncand,), f32),
            pltpu.VMEM((528,), f32),
            pltpu.VMEM((48,), f32),
            pltpu.SemaphoreType.DMA,
        ],
    )(char_logits.reshape(-1), ti.reshape(-1), tv.reshape(-1),
      rp.reshape(-1), sp.reshape(-1), stn.reshape(-1), qs.reshape(-1),
      tbl, wc)
    return out_flat.reshape(B, C)


# speedup vs baseline: 3.6510x; 1.0701x over previous
import functools

import jax
import jax.numpy as jnp
from jax import lax
from jax.experimental import pallas as pl
from jax.experimental.pallas import tpu as pltpu
from jax.experimental.pallas import tpu_sc as plsc


_NEG = -3.0e38
_K = 20
_KPAD = 32
_TW = 256
_RPW = 224
_CHROWS = 8


def _table_kernel(mask_ref, scl_ref, sl_ref, sig_ref, tbl_ref):
    f32 = jnp.float32
    mask = mask_ref[...]
    c, r = mask.shape
    count = jnp.sum(mask, axis=1, keepdims=True)
    countinv = 1.0 / jnp.maximum(count, 1.0)

    sig = sig_ref[...]
    nv = jnp.sqrt(jnp.sum(sig * sig, axis=1, keepdims=True))
    has = (nv > 1e-6).astype(f32)
    signorm = (sig / jnp.maximum(nv, 1e-12)) * has

    tbl_ref[:, 0:r] = mask
    tbl_ref[:, r:224] = jnp.zeros((c, 224 - r), f32)
    tbl_ref[:, 224:225] = countinv
    tbl_ref[:, 225:226] = scl_ref[...].astype(f32)
    tbl_ref[:, 226:227] = sl_ref[...].astype(f32)
    tbl_ref[:, 227:240] = jnp.zeros((c, 13), f32)
    tbl_ref[:, 240:246] = signorm
    tbl_ref[:, 246:256] = jnp.zeros((c, 10), f32)


def _rowprep_kernel(x_ref, rl_ref, st_ref, sc_ref, sty_ref,
                    ti_ref, tv_ref, rp_ref, sp_ref, stn_ref, qs_ref):
    f32 = jnp.float32
    x = x_ref[...]
    bb, c = x.shape

    rp = jax.nn.sigmoid(rl_ref[...])
    total = jnp.sum(rp, axis=1, keepdims=True)
    inv_totclip = 1.0 / jnp.maximum(total, 1e-6)
    rp_ref[:, 0:rp.shape[1]] = rp
    rp_ref[:, rp.shape[1]:] = jnp.zeros((bb, _RPW - rp.shape[1]), f32)

    sp = jax.nn.softmax(st_ref[...], axis=1)
    sp_ref[:, 0:sp.shape[1]] = sp
    sp_ref[:, sp.shape[1]:] = jnp.zeros((bb, 16 - sp.shape[1]), f32)

    sc = sc_ref[...]
    scm = jnp.max(sc, axis=1, keepdims=True)
    sc_iota = lax.broadcasted_iota(jnp.int32, sc.shape, 1)
    pred = jnp.min(jnp.where(sc == scm, sc_iota, jnp.int32(2 ** 30)),
                   axis=1, keepdims=True).astype(f32)

    sty = sty_ref[...]
    stn = sty / jnp.maximum(
        jnp.sqrt(jnp.sum(sty * sty, axis=1, keepdims=True)), 1e-12)
    stn_ref[:, 0:stn.shape[1]] = stn
    stn_ref[:, stn.shape[1]:] = jnp.zeros((bb, 16 - stn.shape[1]), f32)

    nl = 128
    ns_ = (c + nl - 1) // nl
    ninf = jnp.float32(-jnp.inf)
    lane = lax.broadcasted_iota(jnp.int32, (bb, nl), 1)
    m1 = jnp.full((bb, nl), ninf, f32)
    m2, m3, m4 = m1, m1, m1
    g1 = jnp.zeros((bb, nl), jnp.int32)
    g2, g3, g4 = g1, g1, g1
    for s in range(ns_):
        lo = s * nl
        w = min(nl, c - lo)
        xs = x[:, lo:lo + w]
        if w < nl:
            xs = jnp.concatenate(
                [xs, jnp.full((bb, nl - w), ninf, f32)], axis=1)
        v, gv = xs, lane + lo
        c1 = v > m1
        t = jnp.minimum(m1, v)
        m1 = jnp.maximum(m1, v)
        gt = jnp.where(c1, g1, gv)
        g1 = jnp.where(c1, gv, g1)
        v, gv = t, gt
        c2 = v > m2
        t = jnp.minimum(m2, v)
        m2 = jnp.maximum(m2, v)
        gt = jnp.where(c2, g2, gv)
        g2 = jnp.where(c2, gv, g2)
        v, gv = t, gt
        c3 = v > m3
        t = jnp.minimum(m3, v)
        m3 = jnp.maximum(m3, v)
        gt = jnp.where(c3, g3, gv)
        g3 = jnp.where(c3, gv, g3)
        v, gv = t, gt
        c4 = v > m4
        m4 = jnp.maximum(m4, v)
        g4 = jnp.where(c4, gv, g4)

    m0 = jnp.max(m1, axis=1, keepdims=True)
    se = jnp.sum(jnp.exp(x - m0), axis=1, keepdims=True)
    qs_ref[...] = jnp.concatenate(
        [total, inv_totclip, pred, m0, 1.0 / se,
         jnp.zeros((bb, 11), f32)], axis=1)

    big = jnp.int32(2 ** 30)
    cur, gc = m1, g1
    lvl = jnp.zeros((bb, nl), jnp.int32)
    failv = jnp.zeros((bb, nl), jnp.bool_)
    tis, tvs = [], []
    for _ in range(_K):
        m = jnp.max(cur, axis=1, keepdims=True)
        sel = cur == m
        gsel = jnp.min(jnp.where(sel, gc, big), axis=1, keepdims=True)
        tis.append(gsel)
        tvs.append(m)
        win = sel & (gc == gsel)
        failv = failv | (win & (lvl >= 3))
        ncur = jnp.where(lvl == 0, m2,
                         jnp.where(lvl == 1, m3,
                                   jnp.where(lvl == 2, m4, ninf)))
        ngc = jnp.where(lvl == 0, g2,
                        jnp.where(lvl == 1, g3,
                                  jnp.where(lvl == 2, g4, 0)))
        cur = jnp.where(win, ncur, cur)
        gc = jnp.where(win, ngc, gc)
        lvl = lvl + win.astype(jnp.int32)
    row_id = (pl.program_id(0) * bb
              + lax.broadcasted_iota(jnp.int32, (bb, _KPAD - _K), 0))
    pad_ids = ((row_id * (_KPAD - _K)
                + lax.broadcasted_iota(jnp.int32, (bb, _KPAD - _K), 1))
               % jnp.int32(c))
    ti_ref[...] = jnp.concatenate(tis + [pad_ids], axis=1)
    tv_ref[...] = jnp.concatenate(tvs + [jnp.zeros((bb, _KPAD - _K), f32)],
                                  axis=1)

    @pl.when(jnp.any(failv))
    def _redo():
        iota = lax.broadcasted_iota(jnp.int32, (bb, c), 1)
        work = x
        for k in range(_K):
            mm = jnp.max(work, axis=1, keepdims=True)
            eq = work == mm
            il = jnp.min(jnp.where(eq, iota, big), axis=1, keepdims=True)
            oneh = iota == il
            ti_ref[:, k:k + 1] = il
            tv_ref[:, k:k + 1] = mm
            work = jnp.where(oneh, ninf, work)


def _sc_kernel(char_hbm, ti_hbm, tv_hbm, rp_hbm, sp_hbm, stn_hbm, qs_hbm,
               tbl_hbm, wc_hbm, out_hbm,
               chunk_v, ti_v, tv_v, rp_v, sp_v, stn_v, qs_v, gidx_v, gath_v,
               vals_v, wc_v, red_v, sem, *, n_rows, c_dim):
    f32 = jnp.float32
    nc = 2
    wid = lax.axis_index("s") * nc + lax.axis_index("c")
    rows_per_w = n_rows // 32
    nchunks = rows_per_w // _CHROWS
    chunk_words = _CHROWS * c_dim
    ncand = _CHROWS * _KPAD

    pltpu.sync_copy(wc_hbm, wc_v)

    iota16 = lax.iota(jnp.int32, 16)
    mask0 = iota16 == 0
    mask4 = iota16 < (_K - 16)
    cv = wc_v[pl.ds(512, 16)]
    b2s = cv[0]
    rws = cv[1]
    rot_idx = [(iota16 + sh) & 15 for sh in (8, 4, 2, 1)]

    def vsum16(v, base=0):
        for idx in rot_idx:
            red_v[pl.ds(base, 16)] = v
            v = v + plsc.load_gather(red_v.at[pl.ds(base, 16)], [idx])
        return v

    def vsum16x2(va, vb):
        for idx in rot_idx:
            red_v[pl.ds(0, 16)] = va
            red_v[pl.ds(16, 16)] = vb
            va = va + plsc.load_gather(red_v.at[pl.ds(0, 16)], [idx])
            vb = vb + plsc.load_gather(red_v.at[pl.ds(16, 16)], [idx])
        return va, vb

    def do_chunk(ch, carry):
        row0 = wid * rows_per_w + ch * _CHROWS
        base = row0 * c_dim
        pltpu.sync_copy(char_hbm.at[pl.ds(base, chunk_words)], chunk_v)
        pltpu.sync_copy(ti_hbm.at[pl.ds(row0 * _KPAD, ncand)], ti_v)
        pltpu.sync_copy(tv_hbm.at[pl.ds(row0 * _KPAD, ncand)], tv_v)
        pltpu.sync_copy(rp_hbm.at[pl.ds(row0 * _RPW, _CHROWS * _RPW)], rp_v)
        pltpu.sync_copy(sp_hbm.at[pl.ds(row0 * 16, _CHROWS * 16)], sp_v)
        pltpu.sync_copy(stn_hbm.at[pl.ds(row0 * 16, _CHROWS * 16)], stn_v)
        pltpu.sync_copy(qs_hbm.at[pl.ds(row0 * 16, _CHROWS * 16)], qs_v)

        for g in range(_CHROWS):
            t1 = ti_v[pl.ds(g * _KPAD, 16)]
            plsc.store_scatter(gidx_v, [g * _K + iota16], t1)
            t2 = ti_v[pl.ds(g * _KPAD + 16, 16)]
            plsc.store_scatter(gidx_v, [g * _K + 16 + iota16], t2, mask=mask4)
        cp1 = pltpu.async_copy(tbl_hbm.at[gidx_v.at[pl.ds(0, 128)]],
                               gath_v.at[pl.ds(0, 128), :], sem)
        cp2 = pltpu.async_copy(tbl_hbm.at[gidx_v.at[pl.ds(128, 32)]],
                               gath_v.at[pl.ds(128, 32), :], sem)
        cp1.wait()
        cp2.wait()

        def do_row(r8, carry2):
            rbase = r8 * _RPW
            qv = qs_v[pl.ds(r8 * 16, 16)]
            total = qv[0]
            inv_totclip = qv[1]
            pred = qv[2]
            m0 = qv[3]
            inv_se = qv[4]
            stn16 = stn_v[pl.ds(r8 * 16, 16)]

            def do_cand(o, carry3):
                slot = r8 * _KPAD + o
                gv = gath_v.at[r8 * _K + o]
                accs = [rp_v[pl.ds(rbase + j * 16, 16)] * gv[pl.ds(j * 16, 16)]
                        for j in range(4)]
                for j in range(4, _RPW // 16):
                    accs[j % 4] = accs[j % 4] + (
                        rp_v[pl.ds(rbase + j * 16, 16)] * gv[pl.ds(j * 16, 16)])
                acc = (accs[0] + accs[1]) + (accs[2] + accs[3])

                sv = gv[pl.ds(224, 16)]
                countinv = sv[0]
                sclf = sv[1]
                slf = sv[2].astype(jnp.int32)
                sig = gv[pl.ds(240, 16)]
                detv, cosv = vsum16x2(acc, sig * stn16)
                det = detv[0]
                cos = cosv[0]

                f3v = plsc.load_gather(
                    sp_v, [jnp.full((16,), r8 * 16 + slf, jnp.int32)])
                tvv = plsc.load_gather(
                    tv_v, [jnp.full((16,), slot, jnp.int32)])
                tval = tvv[0]

                f1 = det * countinv
                f2 = (total - det) * inv_totclip
                f4 = jnp.abs(pred - sclf) * (1.0 / 29.0)
                f6v = jnp.exp(tvv - m0) * inv_se

                fs = [jnp.full((16,), f1, f32), jnp.full((16,), f2, f32),
                      f3v, jnp.full((16,), f4, f32),
                      jnp.full((16,), cos, f32), f6v]
                sacc = jnp.zeros((16,), f32)
                for t in range(4):
                    h = wc_v[pl.ds(384 + t * 16, 16)]
                    for j in range(6):
                        h = h + fs[j] * wc_v[pl.ds(j * 64 + t * 16, 16)]
                    h = jnp.maximum(h, 0.0)
                    sacc = sacc + h * wc_v[pl.ds(448 + t * 16, 16)]
                score = vsum16(sacc, 32)[0] + b2s
                val = tval + rws * score

                plsc.store_scatter(vals_v, [jnp.full((16,), slot, jnp.int32)],
                                   jnp.full((16,), val, f32), mask=mask0)
                return carry3

            lax.fori_loop(0, _K, do_cand, 0)
            return carry2

        lax.fori_loop(0, _CHROWS, do_row, 0)

        for g in range(_CHROWS):
            pos1 = ti_v[pl.ds(g * _KPAD, 16)] + g * c_dim
            plsc.store_scatter(chunk_v, [pos1], vals_v[pl.ds(g * _KPAD, 16)])
            pos2 = ti_v[pl.ds(g * _KPAD + 16, 16)] + g * c_dim
            plsc.store_scatter(chunk_v, [pos2],
                               vals_v[pl.ds(g * _KPAD + 16, 16)], mask=mask4)

        pltpu.sync_copy(chunk_v, out_hbm.at[pl.ds(base, chunk_words)])
        return carry

    lax.fori_loop(0, nchunks, do_chunk, 0)


@jax.jit
def kernel(char_logits, radical_logits, structure, stroke_count, stroke_types,
           radical_mask, structure_label, stroke_count_label, stroke_type_sig,
           W1, b1, W2, b2, reranker_weight):
    f32 = jnp.float32
    B, C = char_logits.shape
    R = radical_mask.shape[1]
    NS = structure.shape[1]
    NT = stroke_types.shape[1]
    H = W1.shape[1]

    tbl = pl.pallas_call(
        _table_kernel,
        out_shape=jax.ShapeDtypeStruct((C, _TW), f32),
    )(radical_mask, stroke_count_label.reshape(C, 1),
      structure_label.reshape(C, 1), stroke_type_sig)

    BB = 128
    ti, tv, rp, sp, stn, qs = pl.pallas_call(
        _rowprep_kernel,
        grid=(B // BB,),
        in_specs=[
            pl.BlockSpec((BB, C), lambda i: (i, 0)),
            pl.BlockSpec((BB, R), lambda i: (i, 0)),
            pl.BlockSpec((BB, NS), lambda i: (i, 0)),
            pl.BlockSpec((BB, stroke_count.shape[1]), lambda i: (i, 0)),
            pl.BlockSpec((BB, NT), lambda i: (i, 0)),
        ],
        out_specs=[
            pl.BlockSpec((BB, _KPAD), lambda i: (i, 0)),
            pl.BlockSpec((BB, _KPAD), lambda i: (i, 0)),
            pl.BlockSpec((BB, _RPW), lambda i: (i, 0)),
            pl.BlockSpec((BB, 16), lambda i: (i, 0)),
            pl.BlockSpec((BB, 16), lambda i: (i, 0)),
            pl.BlockSpec((BB, 16), lambda i: (i, 0)),
        ],
        out_shape=[
            jax.ShapeDtypeStruct((B, _KPAD), jnp.int32),
            jax.ShapeDtypeStruct((B, _KPAD), f32),
            jax.ShapeDtypeStruct((B, _RPW), f32),
            jax.ShapeDtypeStruct((B, 16), f32),
            jax.ShapeDtypeStruct((B, 16), f32),
            jax.ShapeDtypeStruct((B, 16), f32),
        ],
    )(char_logits, radical_logits, structure, stroke_count, stroke_types)

    wc = jnp.concatenate([
        W1.reshape(-1), b1.reshape(-1), W2.reshape(-1),
        b2.reshape(-1), reranker_weight.reshape(-1),
        jnp.zeros((14,), f32)])

    ncand = _CHROWS * _KPAD
    mesh = plsc.VectorSubcoreMesh(core_axis_name="c", subcore_axis_name="s")
    out_flat = pl.kernel(
        functools.partial(_sc_kernel, n_rows=B, c_dim=C),
        mesh=mesh,
        compiler_params=pltpu.CompilerParams(
            needs_layout_passes=False, use_tc_tiling_on_sc=False),
        out_type=jax.ShapeDtypeStruct((B * C,), f32),
        scratch_types=[
            pltpu.VMEM((_CHROWS * C,), f32),
            pltpu.VMEM((ncand,), jnp.int32),
            pltpu.VMEM((ncand,), f32),
            pltpu.VMEM((_CHROWS * _RPW,), f32),
            pltpu.VMEM((_CHROWS * 16,), f32),
            pltpu.VMEM((_CHROWS * 16,), f32),
            pltpu.VMEM((_CHROWS * 16,), f32),
            pltpu.VMEM((_CHROWS * _K,), jnp.int32),
            pltpu.VMEM((_CHROWS * _K, _TW), f32),
            pltpu.VMEM((ncand,), f32),
            pltpu.VMEM((528,), f32),
            pltpu.VMEM((48,), f32),
            pltpu.SemaphoreType.DMA,
        ],
    )(char_logits.reshape(-1), ti.reshape(-1), tv.reshape(-1),
      rp.reshape(-1), sp.reshape(-1), stn.reshape(-1), qs.reshape(-1),
      tbl, wc)
    return out_flat.reshape(B, C)


# speedup vs baseline: 3.8072x; 1.0428x over previous
import functools

import jax
import jax.numpy as jnp
from jax import lax
from jax.experimental import pallas as pl
from jax.experimental.pallas import tpu as pltpu
from jax.experimental.pallas import tpu_sc as plsc


_NEG = -3.0e38
_K = 20
_KPAD = 32
_TW = 256
_RPW = 224
_CHROWS = 8


def _table_kernel(mask_ref, scl_ref, sl_ref, sig_ref, tbl_ref):
    f32 = jnp.float32
    mask = mask_ref[...]
    c, r = mask.shape
    count = jnp.sum(mask, axis=1, keepdims=True)
    countinv = 1.0 / jnp.maximum(count, 1.0)

    sig = sig_ref[...]
    nv = jnp.sqrt(jnp.sum(sig * sig, axis=1, keepdims=True))
    has = (nv > 1e-6).astype(f32)
    signorm = (sig / jnp.maximum(nv, 1e-12)) * has

    tbl_ref[:, 0:r] = mask
    tbl_ref[:, r:224] = jnp.zeros((c, 224 - r), f32)
    tbl_ref[:, 224:225] = countinv
    tbl_ref[:, 225:226] = scl_ref[...].astype(f32)
    tbl_ref[:, 226:227] = sl_ref[...].astype(f32)
    tbl_ref[:, 227:240] = jnp.zeros((c, 13), f32)
    tbl_ref[:, 240:246] = signorm
    tbl_ref[:, 246:256] = jnp.zeros((c, 10), f32)


def _rowprep_kernel(x_ref, rl_ref, st_ref, sc_ref, sty_ref,
                    ti_ref, tv_ref, rp_ref, sp_ref, stn_ref, qs_ref):
    f32 = jnp.float32
    x = x_ref[...]
    bb, c = x.shape

    rp = jax.nn.sigmoid(rl_ref[...])
    total = jnp.sum(rp, axis=1, keepdims=True)
    inv_totclip = 1.0 / jnp.maximum(total, 1e-6)
    rp_ref[:, 0:rp.shape[1]] = rp
    rp_ref[:, rp.shape[1]:] = jnp.zeros((bb, _RPW - rp.shape[1]), f32)

    sp = jax.nn.softmax(st_ref[...], axis=1)
    sp_ref[:, 0:sp.shape[1]] = sp
    sp_ref[:, sp.shape[1]:] = jnp.zeros((bb, 16 - sp.shape[1]), f32)

    sc = sc_ref[...]
    scm = jnp.max(sc, axis=1, keepdims=True)
    sc_iota = lax.broadcasted_iota(jnp.int32, sc.shape, 1)
    pred = jnp.min(jnp.where(sc == scm, sc_iota, jnp.int32(2 ** 30)),
                   axis=1, keepdims=True).astype(f32)

    sty = sty_ref[...]
    stn = sty / jnp.maximum(
        jnp.sqrt(jnp.sum(sty * sty, axis=1, keepdims=True)), 1e-12)
    stn_ref[:, 0:stn.shape[1]] = stn
    stn_ref[:, stn.shape[1]:] = jnp.zeros((bb, 16 - stn.shape[1]), f32)

    nl = 128
    ns_ = (c + nl - 1) // nl
    ninf = jnp.float32(-jnp.inf)
    lane = lax.broadcasted_iota(jnp.int32, (bb, nl), 1)
    m1 = jnp.full((bb, nl), ninf, f32)
    m2, m3, m4 = m1, m1, m1
    g1 = jnp.zeros((bb, nl), jnp.int32)
    g2, g3, g4 = g1, g1, g1
    for s in range(ns_):
        lo = s * nl
        w = min(nl, c - lo)
        xs = x[:, lo:lo + w]
        if w < nl:
            xs = jnp.concatenate(
                [xs, jnp.full((bb, nl - w), ninf, f32)], axis=1)
        v, gv = xs, lane + lo
        c1 = v > m1
        t = jnp.minimum(m1, v)
        m1 = jnp.maximum(m1, v)
        gt = jnp.where(c1, g1, gv)
        g1 = jnp.where(c1, gv, g1)
        v, gv = t, gt
        c2 = v > m2
        t = jnp.minimum(m2, v)
        m2 = jnp.maximum(m2, v)
        gt = jnp.where(c2, g2, gv)
        g2 = jnp.where(c2, gv, g2)
        v, gv = t, gt
        c3 = v > m3
        t = jnp.minimum(m3, v)
        m3 = jnp.maximum(m3, v)
        gt = jnp.where(c3, g3, gv)
        g3 = jnp.where(c3, gv, g3)
        v, gv = t, gt
        c4 = v > m4
        m4 = jnp.maximum(m4, v)
        g4 = jnp.where(c4, gv, g4)

    m0 = jnp.max(m1, axis=1, keepdims=True)
    se = jnp.sum(jnp.exp(x - m0), axis=1, keepdims=True)
    qs_ref[...] = jnp.concatenate(
        [total, inv_totclip, pred, m0, 1.0 / se,
         jnp.zeros((bb, 11), f32)], axis=1)

    big = jnp.int32(2 ** 30)
    cur, gc = m1, g1
    lvl = jnp.zeros((bb, nl), jnp.int32)
    failv = jnp.zeros((bb, nl), jnp.bool_)
    tis, tvs = [], []
    for _ in range(_K):
        m = jnp.max(cur, axis=1, keepdims=True)
        sel = cur == m
        gsel = jnp.min(jnp.where(sel, gc, big), axis=1, keepdims=True)
        tis.append(gsel)
        tvs.append(m)
        win = sel & (gc == gsel)
        failv = failv | (win & (lvl >= 3))
        ncur = jnp.where(lvl == 0, m2,
                         jnp.where(lvl == 1, m3,
                                   jnp.where(lvl == 2, m4, ninf)))
        ngc = jnp.where(lvl == 0, g2,
                        jnp.where(lvl == 1, g3,
                                  jnp.where(lvl == 2, g4, 0)))
        cur = jnp.where(win, ncur, cur)
        gc = jnp.where(win, ngc, gc)
        lvl = lvl + win.astype(jnp.int32)
    row_id = (pl.program_id(0) * bb
              + lax.broadcasted_iota(jnp.int32, (bb, _KPAD - _K), 0))
    pad_ids = ((row_id * (_KPAD - _K)
                + lax.broadcasted_iota(jnp.int32, (bb, _KPAD - _K), 1))
               % jnp.int32(c))
    ti_ref[...] = jnp.concatenate(tis + [pad_ids], axis=1)
    tv_ref[...] = jnp.concatenate(tvs + [jnp.zeros((bb, _KPAD - _K), f32)],
                                  axis=1)

    @pl.when(jnp.any(failv))
    def _redo():
        iota = lax.broadcasted_iota(jnp.int32, (bb, c), 1)
        work = x
        for k in range(_K):
            mm = jnp.max(work, axis=1, keepdims=True)
            eq = work == mm
            il = jnp.min(jnp.where(eq, iota, big), axis=1, keepdims=True)
            oneh = iota == il
            ti_ref[:, k:k + 1] = il
            tv_ref[:, k:k + 1] = mm
            work = jnp.where(oneh, ninf, work)


def _sc_kernel(char_hbm, ti_hbm, tv_hbm, rp_hbm, sp_hbm, stn_hbm, qs_hbm,
               tbl_hbm, wc_hbm, out_hbm,
               chunk_v, ti_v, tv_v, rp_v, sp_v, stn_v, qs_v, gidx_v, gath_v,
               vals_v, wc_v, red_v, sem, sem2, sem3, *, n_rows, c_dim):
    f32 = jnp.float32
    nc = 2
    wid = lax.axis_index("s") * nc + lax.axis_index("c")
    rows_per_w = n_rows // 32
    nchunks = rows_per_w // _CHROWS
    chunk_words = _CHROWS * c_dim
    ncand = _CHROWS * _KPAD

    pltpu.sync_copy(wc_hbm, wc_v)

    iota16 = lax.iota(jnp.int32, 16)
    mask0 = iota16 == 0
    mask4 = iota16 < (_K - 16)
    cv = wc_v[pl.ds(512, 16)]
    b2s = cv[0]
    rws = cv[1]
    rot_idx = [(iota16 + sh) & 15 for sh in (8, 4, 2, 1)]

    def vsum16(v, base=0):
        for idx in rot_idx:
            red_v[pl.ds(base, 16)] = v
            v = v + plsc.load_gather(red_v.at[pl.ds(base, 16)], [idx])
        return v

    def vsum16x2(va, vb):
        for idx in rot_idx:
            red_v[pl.ds(0, 16)] = va
            red_v[pl.ds(16, 16)] = vb
            va = va + plsc.load_gather(red_v.at[pl.ds(0, 16)], [idx])
            vb = vb + plsc.load_gather(red_v.at[pl.ds(16, 16)], [idx])
        return va, vb

    def do_chunk(ch, carry):
        row0 = wid * rows_per_w + ch * _CHROWS
        base = row0 * c_dim
        a1 = pltpu.async_copy(ti_hbm.at[pl.ds(row0 * _KPAD, ncand)], ti_v,
                              sem2)
        a2 = pltpu.async_copy(tv_hbm.at[pl.ds(row0 * _KPAD, ncand)], tv_v,
                              sem2)
        a3 = pltpu.async_copy(rp_hbm.at[pl.ds(row0 * _RPW, _CHROWS * _RPW)],
                              rp_v, sem2)
        a4 = pltpu.async_copy(sp_hbm.at[pl.ds(row0 * 16, _CHROWS * 16)],
                              sp_v, sem2)
        a5 = pltpu.async_copy(stn_hbm.at[pl.ds(row0 * 16, _CHROWS * 16)],
                              stn_v, sem2)
        a6 = pltpu.async_copy(qs_hbm.at[pl.ds(row0 * 16, _CHROWS * 16)],
                              qs_v, sem2)

        @pl.when(ch > 0)
        def _drain_prev():
            pltpu.make_async_copy(
                chunk_v,
                out_hbm.at[pl.ds(base - chunk_words, chunk_words)],
                sem3).wait()

        pltpu.sync_copy(char_hbm.at[pl.ds(base, chunk_words)], chunk_v)
        a1.wait()
        a2.wait()
        a3.wait()
        a4.wait()
        a5.wait()
        a6.wait()

        for g in range(_CHROWS):
            t1 = ti_v[pl.ds(g * _KPAD, 16)]
            plsc.store_scatter(gidx_v, [g * _K + iota16], t1)
            t2 = ti_v[pl.ds(g * _KPAD + 16, 16)]
            plsc.store_scatter(gidx_v, [g * _K + 16 + iota16], t2, mask=mask4)
        cp1 = pltpu.async_copy(tbl_hbm.at[gidx_v.at[pl.ds(0, 128)]],
                               gath_v.at[pl.ds(0, 128), :], sem)
        cp2 = pltpu.async_copy(tbl_hbm.at[gidx_v.at[pl.ds(128, 32)]],
                               gath_v.at[pl.ds(128, 32), :], sem)
        cp1.wait()
        cp2.wait()

        def do_row(r8, carry2):
            rbase = r8 * _RPW
            qv = qs_v[pl.ds(r8 * 16, 16)]
            total = qv[0]
            inv_totclip = qv[1]
            pred = qv[2]
            m0 = qv[3]
            inv_se = qv[4]
            stn16 = stn_v[pl.ds(r8 * 16, 16)]

            def do_cand(o, carry3):
                slot = r8 * _KPAD + o
                gv = gath_v.at[r8 * _K + o]
                accs = [rp_v[pl.ds(rbase + j * 16, 16)] * gv[pl.ds(j * 16, 16)]
                        for j in range(4)]
                for j in range(4, _RPW // 16):
                    accs[j % 4] = accs[j % 4] + (
                        rp_v[pl.ds(rbase + j * 16, 16)] * gv[pl.ds(j * 16, 16)])
                acc = (accs[0] + accs[1]) + (accs[2] + accs[3])

                sv = gv[pl.ds(224, 16)]
                countinv = sv[0]
                sclf = sv[1]
                slf = sv[2].astype(jnp.int32)
                sig = gv[pl.ds(240, 16)]
                detv, cosv = vsum16x2(acc, sig * stn16)
                det = detv[0]
                cos = cosv[0]

                f3v = plsc.load_gather(
                    sp_v, [jnp.full((16,), r8 * 16 + slf, jnp.int32)])
                tvv = plsc.load_gather(
                    tv_v, [jnp.full((16,), slot, jnp.int32)])
                tval = tvv[0]

                f1 = det * countinv
                f2 = (total - det) * inv_totclip
                f4 = jnp.abs(pred - sclf) * (1.0 / 29.0)
                f6v = jnp.exp(tvv - m0) * inv_se

                fs = [jnp.full((16,), f1, f32), jnp.full((16,), f2, f32),
                      f3v, jnp.full((16,), f4, f32),
                      jnp.full((16,), cos, f32), f6v]
                sacc = jnp.zeros((16,), f32)
                for t in range(4):
                    h = wc_v[pl.ds(384 + t * 16, 16)]
                    for j in range(6):
                        h = h + fs[j] * wc_v[pl.ds(j * 64 + t * 16, 16)]
                    h = jnp.maximum(h, 0.0)
                    sacc = sacc + h * wc_v[pl.ds(448 + t * 16, 16)]
                score = vsum16(sacc, 32)[0] + b2s
                val = tval + rws * score

                plsc.store_scatter(vals_v, [jnp.full((16,), slot, jnp.int32)],
                                   jnp.full((16,), val, f32), mask=mask0)
                return carry3

            lax.fori_loop(0, _K, do_cand, 0)
            return carry2

        lax.fori_loop(0, _CHROWS, do_row, 0)

        for g in range(_CHROWS):
            pos1 = ti_v[pl.ds(g * _KPAD, 16)] + g * c_dim
            plsc.store_scatter(chunk_v, [pos1], vals_v[pl.ds(g * _KPAD, 16)])
            pos2 = ti_v[pl.ds(g * _KPAD + 16, 16)] + g * c_dim
            plsc.store_scatter(chunk_v, [pos2],
                               vals_v[pl.ds(g * _KPAD + 16, 16)], mask=mask4)

        pltpu.async_copy(chunk_v, out_hbm.at[pl.ds(base, chunk_words)], sem3)
        return carry

    lax.fori_loop(0, nchunks, do_chunk, 0)
    last_base = (wid * rows_per_w + (nchunks - 1) * _CHROWS) * c_dim
    pltpu.make_async_copy(
        chunk_v, out_hbm.at[pl.ds(last_base, chunk_words)], sem3).wait()


@jax.jit
def kernel(char_logits, radical_logits, structure, stroke_count, stroke_types,
           radical_mask, structure_label, stroke_count_label, stroke_type_sig,
           W1, b1, W2, b2, reranker_weight):
    f32 = jnp.float32
    B, C = char_logits.shape
    R = radical_mask.shape[1]
    NS = structure.shape[1]
    NT = stroke_types.shape[1]
    H = W1.shape[1]

    tbl = pl.pallas_call(
        _table_kernel,
        out_shape=jax.ShapeDtypeStruct((C, _TW), f32),
    )(radical_mask, stroke_count_label.reshape(C, 1),
      structure_label.reshape(C, 1), stroke_type_sig)

    BB = 256
    ti, tv, rp, sp, stn, qs = pl.pallas_call(
        _rowprep_kernel,
        grid=(B // BB,),
        in_specs=[
            pl.BlockSpec((BB, C), lambda i: (i, 0)),
            pl.BlockSpec((BB, R), lambda i: (i, 0)),
            pl.BlockSpec((BB, NS), lambda i: (i, 0)),
            pl.BlockSpec((BB, stroke_count.shape[1]), lambda i: (i, 0)),
            pl.BlockSpec((BB, NT), lambda i: (i, 0)),
        ],
        out_specs=[
            pl.BlockSpec((BB, _KPAD), lambda i: (i, 0)),
            pl.BlockSpec((BB, _KPAD), lambda i: (i, 0)),
            pl.BlockSpec((BB, _RPW), lambda i: (i, 0)),
            pl.BlockSpec((BB, 16), lambda i: (i, 0)),
            pl.BlockSpec((BB, 16), lambda i: (i, 0)),
            pl.BlockSpec((BB, 16), lambda i: (i, 0)),
        ],
        out_shape=[
            jax.ShapeDtypeStruct((B, _KPAD), jnp.int32),
            jax.ShapeDtypeStruct((B, _KPAD), f32),
            jax.ShapeDtypeStruct((B, _RPW), f32),
            jax.ShapeDtypeStruct((B, 16), f32),
            jax.ShapeDtypeStruct((B, 16), f32),
            jax.ShapeDtypeStruct((B, 16), f32),
        ],
    )(char_logits, radical_logits, structure, stroke_count, stroke_types)

    wc = jnp.concatenate([
        W1.reshape(-1), b1.reshape(-1), W2.reshape(-1),
        b2.reshape(-1), reranker_weight.reshape(-1),
        jnp.zeros((14,), f32)])

    ncand = _CHROWS * _KPAD
    mesh = plsc.VectorSubcoreMesh(core_axis_name="c", subcore_axis_name="s")
    out_flat = pl.kernel(
        functools.partial(_sc_kernel, n_rows=B, c_dim=C),
        mesh=mesh,
        compiler_params=pltpu.CompilerParams(
            needs_layout_passes=False, use_tc_tiling_on_sc=False),
        out_type=jax.ShapeDtypeStruct((B * C,), f32),
        scratch_types=[
            pltpu.VMEM((_CHROWS * C,), f32),
            pltpu.VMEM((ncand,), jnp.int32),
            pltpu.VMEM((ncand,), f32),
            pltpu.VMEM((_CHROWS * _RPW,), f32),
            pltpu.VMEM((_CHROWS * 16,), f32),
            pltpu.VMEM((_CHROWS * 16,), f32),
            pltpu.VMEM((_CHROWS * 16,), f32),
            pltpu.VMEM((_CHROWS * _K,), jnp.int32),
            pltpu.VMEM((_CHROWS * _K, _TW), f32),
            pltpu.VMEM((ncand,), f32),
            pltpu.VMEM((528,), f32),
            pltpu.VMEM((48,), f32),
            pltpu.SemaphoreType.DMA,
            pltpu.SemaphoreType.DMA,
            pltpu.SemaphoreType.DMA,
        ],
    )(char_logits.reshape(-1), ti.reshape(-1), tv.reshape(-1),
      rp.reshape(-1), sp.reshape(-1), stn.reshape(-1), qs.reshape(-1),
      tbl, wc)
    return out_flat.reshape(B, C)
